# baseline (speedup 1.0000x reference)
import functools

import jax
import jax.numpy as jnp
from jax import lax
from jax.experimental import pallas as pl
from jax.experimental.pallas import tpu as pltpu
from jax.experimental.pallas import tpu_sc as plsc

BATCH_SIZE = 16384
DIM = 64
NROWS = 1000000
NUM_CORES = 2
NUM_SUBCORES = 16
NUM_WORKERS = NUM_CORES * NUM_SUBCORES
BPW = BATCH_SIZE // NUM_WORKERS
LANES = 16

N_BLOCKS = NROWS // 128
TAIL_LO = N_BLOCKS * 128
BLK_PER_W = 244
CHUNK_BLKS = 16
N_CHUNKS1 = 16
CAP = 704
CCAP = 128
IDXC = 2048
STAGE_ROWS = 16392
DUMMY = 16384

_ST = jax.ShapeDtypeStruct((STAGE_ROWS, 128), jnp.float32)
_DOTS = jax.ShapeDtypeStruct((BATCH_SIZE,), jnp.float32)


def _fill_dummy(ref, n16):
    for v in range(n16):
        ref[pl.ds(v * LANES, LANES)] = jnp.full((LANES,), DUMMY, jnp.int32)


def _stage1_body(ui, pi, ni, uT, iT, utail, itail, st_u, st_p, st_n,
                 idxbuf, mval0, mval1, mval2, mpos0, mpos1, mpos2,
                 cq0, cq1, cpos0, cpos1, band, tailv, stagA, stagB,
                 sem):
    mval = (mval0, mval1, mval2)
    mpos = (mpos0, mpos1, mpos2)
    cq = (cq0, cq1)
    cpos = (cpos0, cpos1)
    wid = lax.axis_index("s") * NUM_CORES + lax.axis_index("c")
    bb = wid * BLK_PER_W + jnp.maximum(wid - 28, 0)
    nblk = BLK_PER_W + jnp.where(wid >= 28, 1, 0)
    lo = bb * 128
    hi = jnp.where(wid == NUM_WORKERS - 1, NROWS, lo + nblk * 128)

    idx_refs = (ui, pi, ni)
    iota = lax.iota(jnp.int32, LANES)

    counts = []
    for l in range(3):
        def kb_step(kb, n, l=l):
            pltpu.sync_copy(idx_refs[l].at[pl.ds(kb * IDXC, IDXC)], idxbuf)

            def v_step(v, n2):
                x = idxbuf[pl.ds(v * LANES, LANES)]
                m = (x >= lo) & (x < hi)
                cnt = jnp.sum(jnp.where(m, 1, 0))

                @pl.when((cnt > 0) & (n2 < CAP - LANES))
                def _():
                    plsc.store_compressed(mval[l].at[pl.ds(n2, LANES)], x, mask=m)
                    pos = kb * IDXC + v * LANES + iota
                    plsc.store_compressed(mpos[l].at[pl.ds(n2, LANES)], pos, mask=m)

                return n2 + cnt

            return lax.fori_loop(0, IDXC // LANES, v_step, n)

        counts.append(lax.fori_loop(0, BATCH_SIZE // IDXC, kb_step, 0))

    st_refs = (st_u, st_p, st_n)

    @pl.when(wid == NUM_WORKERS - 1)
    def _():
        for l in range(3):
            if l == 0:
                pltpu.sync_copy(utail, tailv)
            elif l == 1:
                pltpu.sync_copy(itail, tailv)
            nl = counts[l]

            def mv_step(mv, carry, l=l):
                base_l = mv * LANES + iota
                msk = base_l < nl
                x = plsc.load_gather(mval[l], [base_l], mask=msk)
                s = plsc.load_gather(mpos[l], [base_l], mask=msk)
                m2 = msk & (x >= TAIL_LO)
                nt = jnp.sum(jnp.where(m2, 1, 0))

                @pl.when(nt > 0)
                def _():
                    _fill_dummy(cpos[0], CCAP // LANES)
                    plsc.store_compressed(cpos[0].at[pl.ds(0, LANES)], s, mask=m2)
                    plsc.store_compressed(cq[0].at[pl.ds(0, LANES)],
                                          x - TAIL_LO, mask=m2)
                    qc = cq[0][pl.ds(0, LANES)]
                    tmask = iota < nt
                    for d in range(DIM):
                        dv = jnp.full((LANES,), d, jnp.int32)
                        vals = plsc.load_gather(tailv, [qc, dv], mask=tmask)
                        plsc.store_scatter(stagA, [iota, dv], vals,
                                           mask=tmask)
                    h = pltpu.async_copy(stagA, st_refs[l].at[cpos[0]],
                                        sem)
                    h.wait()
                return carry

            lax.fori_loop(0, (nl + LANES - 1) // LANES, mv_step, 0,
                          unroll=False)

    scans = ((uT, (0,)), (iT, (1, 2)))
    for tT, lists in scans:
        def chunk_step(c, carry, tT=tT, lists=lists):
            clo = lo + c * (CHUNK_BLKS * 128)
            nb = jnp.minimum(CHUNK_BLKS, nblk - c * CHUNK_BLKS)

            ncs = []
            for j, l in enumerate(lists):
                _fill_dummy(cpos[j], CCAP // LANES)
                nl = counts[l]

                def f_step(mv, nc, l=l, j=j):
                    base_l = mv * LANES + iota
                    msk = base_l < nl
                    x = plsc.load_gather(mval[l], [base_l], mask=msk)
                    s = plsc.load_gather(mpos[l], [base_l], mask=msk)
                    m2 = msk & (x >= clo) & (x < clo + nb * 128)
                    cnt = jnp.sum(jnp.where(m2, 1, 0))

                    @pl.when((cnt > 0) & (nc < CCAP - LANES))
                    def _():
                        plsc.store_compressed(cq[j].at[pl.ds(nc, LANES)],
                                              x - clo, mask=m2)
                        plsc.store_compressed(cpos[j].at[pl.ds(nc, LANES)],
                                              s, mask=m2)
                    return nc + cnt

                ncs.append(lax.fori_loop(0, (nl + LANES - 1) // LANES,
                                         f_step, 0))

            any_m = ncs[0] if len(ncs) == 1 else ncs[0] + ncs[1]

            @pl.when(any_m > 0)
            def _():
                for h in range(8):
                    hs = []
                    for b in range(CHUNK_BLKS):
                        off = clo + jnp.minimum(b, nb - 1) * 128
                        off = pl.multiple_of(off, 128)
                        hs.append(pltpu.async_copy(
                            tT.at[pl.ds(h * 8, 8), pl.ds(off, 128)],
                            band.at[b], sem))
                    for hh in hs:
                        hh.wait()
                    for j, l in enumerate(lists):
                        stag = stagA if j == 0 else stagB
                        nc = ncs[j]

                        def e_step(mv, carry2, j=j, stag=stag, nc=nc):
                            base_l = mv * LANES + iota
                            msk = base_l < nc
                            q = plsc.load_gather(cq[j], [base_l], mask=msk)
                            blk = q >> 7
                            il = q & 127
                            for dl in range(8):
                                dv = jnp.full((LANES,), dl, jnp.int32)
                                vals = plsc.load_gather(band, [blk, dv, il],
                                                        mask=msk)
                                plsc.store_scatter(
                                    stag,
                                    [base_l, jnp.full((LANES,), h * 8 + dl,
                                                      jnp.int32)],
                                    vals, mask=msk)
                            return carry2

                        lax.fori_loop(0, (nc + LANES - 1) // LANES, e_step, 0)

                hs2 = []
                for j, l in enumerate(lists):
                    stag = stagA if j == 0 else stagB
                    hs2.append(pltpu.async_copy(
                        stag, st_refs[l].at[cpos[j]], sem))
                for hh in hs2:
                    hh.wait()

            return carry

        lax.fori_loop(0, N_CHUNKS1, chunk_step, 0, unroll=False)


@functools.cache
def _sc_stage1():
    return functools.partial(
        pl.kernel,
        mesh=plsc.VectorSubcoreMesh(core_axis_name="c", subcore_axis_name="s"),
        out_type=(_ST, _ST, _ST),
        scratch_types=[
            pltpu.VMEM((IDXC,), jnp.int32),
            pltpu.VMEM((CAP,), jnp.int32),
            pltpu.VMEM((CAP,), jnp.int32),
            pltpu.VMEM((CAP,), jnp.int32),
            pltpu.VMEM((CAP,), jnp.int32),
            pltpu.VMEM((CAP,), jnp.int32),
            pltpu.VMEM((CAP,), jnp.int32),
            pltpu.VMEM((CCAP,), jnp.int32),
            pltpu.VMEM((CCAP,), jnp.int32),
            pltpu.VMEM((CCAP,), jnp.int32),
            pltpu.VMEM((CCAP,), jnp.int32),
            pltpu.VMEM((CHUNK_BLKS, 8, 128), jnp.float32),
            pltpu.VMEM((DIM, 128), jnp.float32),
            pltpu.VMEM((CCAP, 128), jnp.float32),
            pltpu.VMEM((CCAP, 128), jnp.float32),
            pltpu.SemaphoreType.DMA,
        ],
        compiler_params=pltpu.CompilerParams(needs_layout_passes=False),
    )(_stage1_body)


def _stage2_body(st_u, st_p, st_n, out_up, out_un,
                 ru, rp, rn, acc_up, acc_un, sem):
    wid = lax.axis_index("s") * NUM_CORES + lax.axis_index("c")
    base = wid * BPW
    iota = lax.iota(jnp.int32, LANES)

    def chunk_step(c, carry):
        row0 = base + c * CCAP
        h1 = pltpu.async_copy(st_u.at[pl.ds(row0, CCAP)], ru, sem)
        h2 = pltpu.async_copy(st_p.at[pl.ds(row0, CCAP)], rp, sem)
        h3 = pltpu.async_copy(st_n.at[pl.ds(row0, CCAP)], rn, sem)
        h1.wait()
        h2.wait()
        h3.wait()

        def group_step(g, carry2):
            rows = iota + g * LANES
            aup = jnp.zeros((LANES,), jnp.float32)
            aun = jnp.zeros((LANES,), jnp.float32)
            for d in range(DIM):
                dv = jnp.full((LANES,), d, jnp.int32)
                du = plsc.load_gather(ru, [rows, dv])
                dp = plsc.load_gather(rp, [rows, dv])
                dn = plsc.load_gather(rn, [rows, dv])
                aup = aup + du * dp
                aun = aun + du * dn
            off = c * CCAP + g * LANES
            acc_up[pl.ds(off, LANES)] = aup
            acc_un[pl.ds(off, LANES)] = aun
            return carry2

        lax.fori_loop(0, CCAP // LANES, group_step, 0)
        return carry

    lax.fori_loop(0, BPW // CCAP, chunk_step, 0)
    pltpu.sync_copy(acc_up, out_up.at[pl.ds(base, BPW)])
    pltpu.sync_copy(acc_un, out_un.at[pl.ds(base, BPW)])


@functools.cache
def _sc_stage2():
    return functools.partial(
        pl.kernel,
        mesh=plsc.VectorSubcoreMesh(core_axis_name="c", subcore_axis_name="s"),
        out_type=(_DOTS, _DOTS),
        scratch_types=[
            pltpu.VMEM((CCAP, 128), jnp.float32),
            pltpu.VMEM((CCAP, 128), jnp.float32),
            pltpu.VMEM((CCAP, 128), jnp.float32),
            pltpu.VMEM((BPW,), jnp.float32),
            pltpu.VMEM((BPW,), jnp.float32),
            pltpu.SemaphoreType.DMA,
        ],
        compiler_params=pltpu.CompilerParams(needs_layout_passes=False),
    )(_stage2_body)


def _loss_body(up_ref, un_ref, o_ref):
    d = jax.nn.sigmoid(up_ref[...]) - jax.nn.sigmoid(un_ref[...])
    o_ref[0, 0] = jnp.sum(-jax.nn.log_sigmoid(d)) * (1.0 / BATCH_SIZE)


_tc_loss = pl.pallas_call(
    _loss_body,
    out_specs=pl.BlockSpec(memory_space=pltpu.SMEM),
    out_shape=jax.ShapeDtypeStruct((1, 1), jnp.float32),
)


def kernel(user_indices, pos_item_indices, neg_item_indices,
           user_embedding, item_embedding):
    ui = user_indices.astype(jnp.int32)
    pi = pos_item_indices.astype(jnp.int32)
    ni = neg_item_indices.astype(jnp.int32)
    utail = jnp.pad(user_embedding[TAIL_LO:], ((0, 0), (0, 128 - DIM)))
    itail = jnp.pad(item_embedding[TAIL_LO:], ((0, 0), (0, 128 - DIM)))
    st_u, st_p, st_n = _sc_stage1()(
        ui, pi, ni, user_embedding.T, item_embedding.T, utail, itail)
    up, un = _sc_stage2()(st_u, st_p, st_n)
    out = _tc_loss(up.reshape(128, 128), un.reshape(128, 128))
    return out[0, 0]


# speedup vs baseline: 1.0129x; 1.0129x over previous
import functools

import jax
import jax.numpy as jnp
from jax import lax
from jax.experimental import pallas as pl
from jax.experimental.pallas import tpu as pltpu
from jax.experimental.pallas import tpu_sc as plsc

BATCH_SIZE = 16384
DIM = 64
NROWS = 1000000
NUM_CORES = 2
NUM_SUBCORES = 16
NUM_WORKERS = NUM_CORES * NUM_SUBCORES
BPW = BATCH_SIZE // NUM_WORKERS
LANES = 16

N_BLOCKS = NROWS // 128
TAIL_LO = N_BLOCKS * 128
BLK_PER_W = 244
CHUNK_BLKS = 16
N_CHUNKS1 = 16
CAP = 704
CCAP = 128
IDXC = 2048
STAGE_ROWS = 16392
DUMMY = 16384

_ST = jax.ShapeDtypeStruct((STAGE_ROWS, 128), jnp.float32)
_DOTS = jax.ShapeDtypeStruct((BATCH_SIZE,), jnp.float32)


def _fill_dummy(ref, n16):
    for v in range(n16):
        ref[pl.ds(v * LANES, LANES)] = jnp.full((LANES,), DUMMY, jnp.int32)


def _stage1_body(ui, pi, ni, uT, iT, utail, itail, st_u, st_p, st_n,
                 idxbuf, mval0, mval1, mval2, mpos0, mpos1, mpos2,
                 cq0, cpos0, cqf, cpf, ncb, spos, band, tailv, stagA, stagB,
                 sem, sem2):
    mval = (mval0, mval1, mval2)
    mpos = (mpos0, mpos1, mpos2)
    cq = (cq0,)
    cpos = (cpos0,)
    wid = lax.axis_index("s") * NUM_CORES + lax.axis_index("c")
    bb = wid * BLK_PER_W + jnp.maximum(wid - 28, 0)
    nblk = BLK_PER_W + jnp.where(wid >= 28, 1, 0)
    lo = bb * 128
    hi = jnp.where(wid == NUM_WORKERS - 1, NROWS, lo + nblk * 128)

    idx_refs = (ui, pi, ni)
    iota = lax.iota(jnp.int32, LANES)

    counts = []
    for l in range(3):
        def kb_step(kb, n, l=l):
            pltpu.sync_copy(idx_refs[l].at[pl.ds(kb * IDXC, IDXC)], idxbuf)

            def v_step(v, n2):
                x = idxbuf[pl.ds(v * LANES, LANES)]
                m = (x >= lo) & (x < hi)
                cnt = jnp.sum(jnp.where(m, 1, 0))

                @pl.when((cnt > 0) & (n2 < CAP - LANES))
                def _():
                    plsc.store_compressed(mval[l].at[pl.ds(n2, LANES)], x, mask=m)
                    pos = kb * IDXC + v * LANES + iota
                    plsc.store_compressed(mpos[l].at[pl.ds(n2, LANES)], pos, mask=m)

                return n2 + cnt

            return lax.fori_loop(0, IDXC // LANES, v_step, n)

        counts.append(lax.fori_loop(0, BATCH_SIZE // IDXC, kb_step, 0))

    st_refs = (st_u, st_p, st_n)

    @pl.when(wid == NUM_WORKERS - 1)
    def _():
        for l in range(3):
            if l == 0:
                pltpu.sync_copy(utail, tailv)
            elif l == 1:
                pltpu.sync_copy(itail, tailv)
            nl = counts[l]

            def mv_step(mv, carry, l=l):
                base_l = mv * LANES + iota
                msk = base_l < nl
                x = plsc.load_gather(mval[l], [base_l], mask=msk)
                s = plsc.load_gather(mpos[l], [base_l], mask=msk)
                m2 = msk & (x >= TAIL_LO)
                nt = jnp.sum(jnp.where(m2, 1, 0))

                @pl.when(nt > 0)
                def _():
                    _fill_dummy(cpos[0], CCAP // LANES)
                    plsc.store_compressed(cpos[0].at[pl.ds(0, LANES)], s, mask=m2)
                    plsc.store_compressed(cq[0].at[pl.ds(0, LANES)],
                                          x - TAIL_LO, mask=m2)
                    qc = cq[0][pl.ds(0, LANES)]
                    tmask = iota < nt
                    for d in range(DIM):
                        dv = jnp.full((LANES,), d, jnp.int32)
                        vals = plsc.load_gather(tailv, [qc, dv], mask=tmask)
                        plsc.store_scatter(stagA, [iota, dv], vals,
                                           mask=tmask)
                    h = pltpu.async_copy(stagA, st_refs[l].at[cpos[0]],
                                        sem)
                    h.wait()
                return carry

            lax.fori_loop(0, (nl + LANES - 1) // LANES, mv_step, 0,
                          unroll=False)

    CW = CHUNK_BLKS * 128
    scans = ((uT, (0,)), (iT, (1, 2)))
    for tT, lists in scans:
        for j, l in enumerate(lists):
            def fill_step(v, carry, j=j):
                cpf[pl.ds(j * 16 * CCAP + v * LANES, LANES)] = jnp.full(
                    (LANES,), DUMMY, jnp.int32)
                return carry
            lax.fori_loop(0, 16 * CCAP // LANES, fill_step, 0)
            nl = counts[l]

            def pchunk(c, carry, j=j, l=l):
                cstart = lo + jnp.minimum(c * CW, nblk * 128 - CW)
                fbase = (j * 16 + c) * CCAP

                def f_step(mv, nc, l=l):
                    base_l = mv * LANES + iota
                    msk = base_l < nl
                    x = plsc.load_gather(mval[l], [base_l], mask=msk)
                    s = plsc.load_gather(mpos[l], [base_l], mask=msk)
                    m2 = msk & (x >= cstart) & (x < cstart + CW)
                    cnt = jnp.sum(jnp.where(m2, 1, 0))

                    @pl.when((cnt > 0) & (nc < CCAP - LANES))
                    def _():
                        plsc.store_compressed(
                            cqf.at[pl.ds(fbase + nc, LANES)], x - cstart,
                            mask=m2)
                        plsc.store_compressed(
                            cpf.at[pl.ds(fbase + nc, LANES)], s, mask=m2)
                    return nc + cnt

                nc = lax.fori_loop(0, (nl + LANES - 1) // LANES, f_step, 0)
                ncb[pl.ds((j * 16 + c) * LANES, LANES)] = jnp.broadcast_to(
                    nc, (LANES,)).astype(jnp.int32)
                return carry

            lax.fori_loop(0, N_CHUNKS1, pchunk, 0)

        def src_at(c, h):
            off = lo + jnp.minimum(c * CW, nblk * 128 - CW)
            off = pl.multiple_of(off, 128)
            return tT.at[pl.ds(h * 8, 8), pl.ds(off, CW)]

        pltpu.async_copy(src_at(0, 0), band.at[0], sem)

        def chunk_step(c, carry, tT=tT, lists=lists):
            for h in range(8):
                g = c * 8 + h
                par = g & 1
                pltpu.make_async_copy(src_at(0, 0), band.at[par], sem).wait()
                if h < 7:
                    pltpu.async_copy(src_at(c, h + 1), band.at[(g + 1) & 1],
                                     sem)
                else:
                    @pl.when(c + 1 < N_CHUNKS1)
                    def _():
                        pltpu.async_copy(src_at(c + 1, 0),
                                         band.at[(g + 1) & 1], sem)
                for j, l in enumerate(lists):
                    stag = stagA if j == 0 else stagB
                    nc = ncb[pl.ds((j * 16 + c) * LANES, LANES)][0]
                    fbase = (j * 16 + c) * CCAP

                    def e_step(mv, carry2, j=j, stag=stag, nc=nc,
                               fbase=fbase, par=par, h=h):
                        base_l = mv * LANES + iota
                        msk = base_l < nc
                        q = plsc.load_gather(cqf, [fbase + base_l], mask=msk)
                        for dl in range(8):
                            dv = jnp.full((LANES,), dl, jnp.int32)
                            vals = plsc.load_gather(band.at[par], [dv, q],
                                                    mask=msk)
                            plsc.store_scatter(
                                stag,
                                [base_l, jnp.full((LANES,), h * 8 + dl,
                                                  jnp.int32)],
                                vals, mask=msk)
                        return carry2

                    lax.fori_loop(0, (nc + LANES - 1) // LANES, e_step, 0)

            hs2 = []
            for j, l in enumerate(lists):
                stag = stagA if j == 0 else stagB
                fbase = (j * 16 + c) * CCAP
                for v in range(CCAP // LANES):
                    spos[pl.ds(v * LANES, LANES)] = cpf[
                        pl.ds(fbase + v * LANES, LANES)]
                hs2.append(pltpu.async_copy(
                    stag, st_refs[l].at[spos], sem2))
            for hh in hs2:
                hh.wait()
            return carry

        lax.fori_loop(0, N_CHUNKS1, chunk_step, 0, unroll=False)


@functools.cache
def _sc_stage1():
    return functools.partial(
        pl.kernel,
        mesh=plsc.VectorSubcoreMesh(core_axis_name="c", subcore_axis_name="s"),
        out_type=(_ST, _ST, _ST),
        scratch_types=[
            pltpu.VMEM((IDXC,), jnp.int32),
            pltpu.VMEM((CAP,), jnp.int32),
            pltpu.VMEM((CAP,), jnp.int32),
            pltpu.VMEM((CAP,), jnp.int32),
            pltpu.VMEM((CAP,), jnp.int32),
            pltpu.VMEM((CAP,), jnp.int32),
            pltpu.VMEM((CAP,), jnp.int32),
            pltpu.VMEM((CCAP,), jnp.int32),
            pltpu.VMEM((CCAP,), jnp.int32),
            pltpu.VMEM((2 * 16 * CCAP,), jnp.int32),
            pltpu.VMEM((2 * 16 * CCAP,), jnp.int32),
            pltpu.VMEM((2 * 16 * LANES,), jnp.int32),
            pltpu.VMEM((CCAP,), jnp.int32),
            pltpu.VMEM((2, 8, CHUNK_BLKS * 128), jnp.float32),
            pltpu.VMEM((DIM, 128), jnp.float32),
            pltpu.VMEM((CCAP, 128), jnp.float32),
            pltpu.VMEM((CCAP, 128), jnp.float32),
            pltpu.SemaphoreType.DMA,
            pltpu.SemaphoreType.DMA,
        ],
        compiler_params=pltpu.CompilerParams(needs_layout_passes=False),
    )(_stage1_body)


def _stage2_body(st_u, st_p, st_n, out_up, out_un,
                 ru, rp, rn, acc_up, acc_un, sem):
    wid = lax.axis_index("s") * NUM_CORES + lax.axis_index("c")
    base = wid * BPW
    iota = lax.iota(jnp.int32, LANES)

    def chunk_step(c, carry):
        row0 = base + c * CCAP
        h1 = pltpu.async_copy(st_u.at[pl.ds(row0, CCAP)], ru, sem)
        h2 = pltpu.async_copy(st_p.at[pl.ds(row0, CCAP)], rp, sem)
        h3 = pltpu.async_copy(st_n.at[pl.ds(row0, CCAP)], rn, sem)
        h1.wait()
        h2.wait()
        h3.wait()

        def group_step(g, carry2):
            rows = iota + g * LANES
            aup = jnp.zeros((LANES,), jnp.float32)
            aun = jnp.zeros((LANES,), jnp.float32)
            for d in range(DIM):
                dv = jnp.full((LANES,), d, jnp.int32)
                du = plsc.load_gather(ru, [rows, dv])
                dp = plsc.load_gather(rp, [rows, dv])
                dn = plsc.load_gather(rn, [rows, dv])
                aup = aup + du * dp
                aun = aun + du * dn
            off = c * CCAP + g * LANES
            acc_up[pl.ds(off, LANES)] = aup
            acc_un[pl.ds(off, LANES)] = aun
            return carry2

        lax.fori_loop(0, CCAP // LANES, group_step, 0)
        return carry

    lax.fori_loop(0, BPW // CCAP, chunk_step, 0)
    pltpu.sync_copy(acc_up, out_up.at[pl.ds(base, BPW)])
    pltpu.sync_copy(acc_un, out_un.at[pl.ds(base, BPW)])


@functools.cache
def _sc_stage2():
    return functools.partial(
        pl.kernel,
        mesh=plsc.VectorSubcoreMesh(core_axis_name="c", subcore_axis_name="s"),
        out_type=(_DOTS, _DOTS),
        scratch_types=[
            pltpu.VMEM((CCAP, 128), jnp.float32),
            pltpu.VMEM((CCAP, 128), jnp.float32),
            pltpu.VMEM((CCAP, 128), jnp.float32),
            pltpu.VMEM((BPW,), jnp.float32),
            pltpu.VMEM((BPW,), jnp.float32),
            pltpu.SemaphoreType.DMA,
        ],
        compiler_params=pltpu.CompilerParams(needs_layout_passes=False),
    )(_stage2_body)


def _loss_body(up_ref, un_ref, o_ref):
    d = jax.nn.sigmoid(up_ref[...]) - jax.nn.sigmoid(un_ref[...])
    o_ref[0, 0] = jnp.sum(-jax.nn.log_sigmoid(d)) * (1.0 / BATCH_SIZE)


_tc_loss = pl.pallas_call(
    _loss_body,
    out_specs=pl.BlockSpec(memory_space=pltpu.SMEM),
    out_shape=jax.ShapeDtypeStruct((1, 1), jnp.float32),
)


def kernel(user_indices, pos_item_indices, neg_item_indices,
           user_embedding, item_embedding):
    ui = user_indices.astype(jnp.int32)
    pi = pos_item_indices.astype(jnp.int32)
    ni = neg_item_indices.astype(jnp.int32)
    utail = jnp.pad(user_embedding[TAIL_LO:], ((0, 0), (0, 128 - DIM)))
    itail = jnp.pad(item_embedding[TAIL_LO:], ((0, 0), (0, 128 - DIM)))
    st_u, st_p, st_n = _sc_stage1()(
        ui, pi, ni, user_embedding.T, item_embedding.T, utail, itail)
    up, un = _sc_stage2()(st_u, st_p, st_n)
    out = _tc_loss(up.reshape(128, 128), un.reshape(128, 128))
    return out[0, 0]


# speedup vs baseline: 4.0577x; 4.0061x over previous
import functools

import jax
import jax.numpy as jnp
from jax import lax
from jax.experimental import pallas as pl
from jax.experimental.pallas import tpu as pltpu
from jax.experimental.pallas import tpu_sc as plsc

BATCH_SIZE = 16384
DIM = 64
NUM_CORES = 2
NUM_SUBCORES = 16
NUM_WORKERS = NUM_CORES * NUM_SUBCORES
BPW = BATCH_SIZE // NUM_WORKERS
CHUNK = 128
N_CHUNKS = BPW // CHUNK
LANES = 16
N_GROUPS = CHUNK // LANES


def _dots_body(jt, uemb, iemb, out_up, out_un,
               jt_v, ru, rp, rn, acc_up, acc_un, sem):
    wid = lax.axis_index("s") * NUM_CORES + lax.axis_index("c")
    base = wid * BPW
    pltpu.sync_copy(jt.at[wid], jt_v)

    def chunk_step(c, carry):
        h1 = pltpu.async_copy(uemb.at[jt_v.at[0, c]], ru, sem)
        h2 = pltpu.async_copy(iemb.at[jt_v.at[1, c]], rp, sem)
        h3 = pltpu.async_copy(iemb.at[jt_v.at[2, c]], rn, sem)
        h1.wait()
        h2.wait()
        h3.wait()

        def group_step(g, carry2):
            rows = lax.iota(jnp.int32, LANES) + g * LANES
            aup = jnp.zeros((LANES,), jnp.float32)
            aun = jnp.zeros((LANES,), jnp.float32)
            for d in range(DIM):
                dvec = jnp.full((LANES,), d, jnp.int32)
                du = plsc.load_gather(ru, [rows, dvec])
                dp = plsc.load_gather(rp, [rows, dvec])
                dn = plsc.load_gather(rn, [rows, dvec])
                aup = aup + du * dp
                aun = aun + du * dn
            off = c * CHUNK + g * LANES
            acc_up[pl.ds(off, LANES)] = aup
            acc_un[pl.ds(off, LANES)] = aun
            return carry2

        lax.fori_loop(0, N_GROUPS, group_step, 0)
        return carry

    lax.fori_loop(0, N_CHUNKS, chunk_step, 0)
    pltpu.sync_copy(acc_up, out_up.at[pl.ds(base, BPW)])
    pltpu.sync_copy(acc_un, out_un.at[pl.ds(base, BPW)])


_DOTS = jax.ShapeDtypeStruct((BATCH_SIZE,), jnp.float32)


@functools.cache
def _sc_dots():
    return functools.partial(
        pl.kernel,
        mesh=plsc.VectorSubcoreMesh(core_axis_name="c", subcore_axis_name="s"),
        out_type=(_DOTS, _DOTS),
        scratch_types=[
            pltpu.VMEM((3, N_CHUNKS, CHUNK), jnp.int32),
            pltpu.VMEM((CHUNK, DIM), jnp.float32),
            pltpu.VMEM((CHUNK, DIM), jnp.float32),
            pltpu.VMEM((CHUNK, DIM), jnp.float32),
            pltpu.VMEM((BPW,), jnp.float32),
            pltpu.VMEM((BPW,), jnp.float32),
            pltpu.SemaphoreType.DMA,
        ],
        compiler_params=pltpu.CompilerParams(
            needs_layout_passes=False, use_tc_tiling_on_sc=False),
    )(_dots_body)


def _loss_body(up_ref, un_ref, o_ref):
    d = jax.nn.sigmoid(up_ref[...]) - jax.nn.sigmoid(un_ref[...])
    o_ref[0, 0] = jnp.sum(-jax.nn.log_sigmoid(d)) * (1.0 / BATCH_SIZE)


_tc_loss = pl.pallas_call(
    _loss_body,
    out_specs=pl.BlockSpec(memory_space=pltpu.SMEM),
    out_shape=jax.ShapeDtypeStruct((1, 1), jnp.float32),
)


def kernel(user_indices, pos_item_indices, neg_item_indices,
           user_embedding, item_embedding):
    idx = jnp.stack([user_indices, pos_item_indices, neg_item_indices])
    idx = idx.astype(jnp.int32)
    jt = idx.reshape(3, NUM_WORKERS, N_CHUNKS, CHUNK).transpose(1, 0, 2, 3)
    up, un = _sc_dots()(jt, user_embedding, item_embedding)
    out = _tc_loss(up.reshape(128, 128), un.reshape(128, 128))
    return out[0, 0]
